# Initial kernel scaffold; baseline (speedup 1.0000x reference)
import functools

import jax
import jax.numpy as jnp
from jax import lax
from jax.experimental import pallas as pl
from jax.experimental.pallas import tpu as pltpu
from jax.experimental.pallas import tpu_sc as plsc

_LANES = 16


@functools.partial(jax.jit, static_argnums=(2, 3, 4, 5))
def _gather_mean(idx3, table, nb, nchunks, rpc, s):
    info = plsc.get_sparse_core_info()
    nc, ns = info.num_cores, info.num_subcores
    nw = nc * ns
    d = table.shape[1]
    bp = nw * nb

    mesh = plsc.VectorSubcoreMesh(core_axis_name="c", subcore_axis_name="s")

    @functools.partial(
        pl.kernel,
        mesh=mesh,
        out_type=jax.ShapeDtypeStruct((bp, d), jnp.float32),
        scratch_types=[
            pltpu.VMEM((nchunks, rpc * s), jnp.int32),
            pltpu.VMEM((rpc * s, d), jnp.float32),
            pltpu.VMEM((nb, d), jnp.float32),
            pltpu.SemaphoreType.DMA,
        ],
    )
    def k(idx_hbm, table_hbm, out_hbm, idx_v, buf_v, out_v, sem):
        w = lax.axis_index("s") * nc + lax.axis_index("c")
        pltpu.sync_copy(idx_hbm.at[w], idx_v)

        scale = jnp.float32(1.0 / s)
        nvec = d // _LANES

        def body(c, carry):
            pltpu.async_copy(table_hbm.at[idx_v.at[c]], buf_v, sem).wait()
            for r in range(rpc):
                base = r * s
                accs = [buf_v[base, pl.ds(v * _LANES, _LANES)]
                        for v in range(nvec)]
                for j in range(1, s):
                    for v in range(nvec):
                        accs[v] = accs[v] + buf_v[base + j,
                                                  pl.ds(v * _LANES, _LANES)]
                row = c * rpc + r
                for v in range(nvec):
                    out_v[row, pl.ds(v * _LANES, _LANES)] = accs[v] * scale
            return carry

        lax.fori_loop(0, nchunks, body, 0)
        pltpu.sync_copy(out_v, out_hbm.at[pl.ds(w * nb, nb)])

    return k(idx3, table)


def kernel(nodes, neighbors, num_sample, emb_weight):
    b, s = neighbors.shape
    idx = neighbors.astype(jnp.int32)

    info = plsc.get_sparse_core_info()
    nw = info.num_cores * info.num_subcores
    rpc = max(1, 128 // s)
    chunk_b = nw * rpc
    bp = ((b + chunk_b - 1) // chunk_b) * chunk_b
    if bp != b:
        idx = jnp.pad(idx, ((0, bp - b), (0, 0)))
    nb = bp // nw
    nchunks = nb // rpc
    idx3 = idx.reshape(nw, nchunks, rpc * s)

    out = _gather_mean(idx3, emb_weight.astype(jnp.float32),
                       nb, nchunks, rpc, s)
    return out[:b]


# speedup vs baseline: 1.3340x; 1.3340x over previous
import functools

import jax
import jax.numpy as jnp
from jax import lax
from jax.experimental import pallas as pl
from jax.experimental.pallas import tpu as pltpu
from jax.experimental.pallas import tpu_sc as plsc

_LANES = 16


@functools.partial(jax.jit, static_argnums=(2, 3, 4, 5))
def _gather_mean(idx3, table, nb, nchunks, rpc, s):
    info = plsc.get_sparse_core_info()
    nc, ns = info.num_cores, info.num_subcores
    nw = nc * ns
    d = table.shape[1]
    bp = nw * nb

    mesh = plsc.VectorSubcoreMesh(core_axis_name="c", subcore_axis_name="s")

    @functools.partial(
        pl.kernel,
        mesh=mesh,
        out_type=jax.ShapeDtypeStruct((bp, d), jnp.float32),
        scratch_types=[
            pltpu.VMEM((nchunks, rpc * s), jnp.int32),
            pltpu.VMEM((rpc * s, d), jnp.float32),
            pltpu.VMEM((nb, d), jnp.float32),
            pltpu.SemaphoreType.DMA,
        ],
    )
    def k(idx_hbm, table_hbm, out_hbm, idx_v, buf_v, out_v, sem):
        w = lax.axis_index("s") * nc + lax.axis_index("c")
        pltpu.sync_copy(idx_hbm.at[w], idx_v)

        scale = jnp.float32(1.0 / s)
        nvec = d // _LANES

        def body(c, carry):
            pltpu.async_copy(table_hbm.at[idx_v.at[c]], buf_v, sem).wait()
            for r in range(rpc):
                base = r * s
                accs = [buf_v[base, pl.ds(v * _LANES, _LANES)]
                        for v in range(nvec)]
                for j in range(1, s):
                    for v in range(nvec):
                        accs[v] = accs[v] + buf_v[base + j,
                                                  pl.ds(v * _LANES, _LANES)]
                row = c * rpc + r
                for v in range(nvec):
                    out_v[row, pl.ds(v * _LANES, _LANES)] = accs[v] * scale
            return carry

        lax.fori_loop(jnp.int32(0), jnp.int32(nchunks), body, jnp.int32(0))
        pltpu.sync_copy(out_v, out_hbm.at[pl.ds(w * nb, nb)])

    return k(idx3, table)


def kernel(nodes, neighbors, num_sample, emb_weight):
    b, s = neighbors.shape
    idx = neighbors.astype(jnp.int32)

    info = plsc.get_sparse_core_info()
    nw = info.num_cores * info.num_subcores
    rpc = max(1, 128 // s)
    align = nw * rpc * 2 if rpc < 8 else nw * rpc
    while align % (nw * 8):
        align *= 2
    bp = ((b + align - 1) // align) * align
    if bp != b:
        idx = jnp.pad(idx, ((0, bp - b), (0, 0)))
    nb = bp // nw
    nchunks = nb // rpc
    idx3 = idx.reshape(nw, nchunks, rpc * s)

    out = _gather_mean(idx3, emb_weight.astype(jnp.float32),
                       nb, nchunks, rpc, s)
    return out[:b]


# speedup vs baseline: 1.5476x; 1.1602x over previous
import functools

import jax
import jax.numpy as jnp
from jax import lax
from jax.experimental import pallas as pl
from jax.experimental.pallas import tpu as pltpu
from jax.experimental.pallas import tpu_sc as plsc

_LANES = 16


@functools.partial(jax.jit, static_argnums=(2, 3, 4, 5))
def _gather_mean(idx3, table, nb, nchunks, rpc, s):
    info = plsc.get_sparse_core_info()
    nc, ns = info.num_cores, info.num_subcores
    nw = nc * ns
    d = table.shape[1]
    bp = nw * nb

    mesh = plsc.VectorSubcoreMesh(core_axis_name="c", subcore_axis_name="s")

    @functools.partial(
        pl.kernel,
        mesh=mesh,
        out_type=jax.ShapeDtypeStruct((bp, d), jnp.float32),
        scratch_types=[
            pltpu.VMEM((nchunks, rpc * s), jnp.int32),
            pltpu.VMEM((rpc * s, d), jnp.float32),
            pltpu.VMEM((rpc * s, d), jnp.float32),
            pltpu.VMEM((nb, d), jnp.float32),
            pltpu.SemaphoreType.DMA,
            pltpu.SemaphoreType.DMA,
        ],
    )
    def k(idx_hbm, table_hbm, out_hbm, idx_v, buf0, buf1, out_v, sem0, sem1):
        w = lax.axis_index("s") * nc + lax.axis_index("c")
        pltpu.sync_copy(idx_hbm.at[w], idx_v)

        scale = jnp.float32(1.0 / s)
        nvec = d // _LANES

        def compute(c, buf):
            for r in range(rpc):
                base = r * s
                accs = [buf[base, pl.ds(v * _LANES, _LANES)]
                        for v in range(nvec)]
                for j in range(1, s):
                    for v in range(nvec):
                        accs[v] = accs[v] + buf[base + j,
                                                pl.ds(v * _LANES, _LANES)]
                row = c * rpc + r
                for v in range(nvec):
                    out_v[row, pl.ds(v * _LANES, _LANES)] = accs[v] * scale

        pltpu.async_copy(table_hbm.at[idx_v.at[jnp.int32(0)]], buf0, sem0)

        def body(t, carry):
            c0 = t * 2
            c1 = c0 + 1
            pltpu.make_async_copy(table_hbm.at[idx_v.at[c0]], buf0, sem0).wait()
            pltpu.async_copy(table_hbm.at[idx_v.at[c1]], buf1, sem1)
            compute(c0, buf0)
            pltpu.make_async_copy(table_hbm.at[idx_v.at[c1]], buf1, sem1).wait()

            @pl.when(c1 + 1 < nchunks)
            def _():
                pltpu.async_copy(table_hbm.at[idx_v.at[c1 + 1]], buf0, sem0)

            compute(c1, buf1)
            return carry

        lax.fori_loop(jnp.int32(0), jnp.int32(nchunks // 2), body,
                      jnp.int32(0))
        pltpu.sync_copy(out_v, out_hbm.at[pl.ds(w * nb, nb)])

    return k(idx3, table)


def kernel(nodes, neighbors, num_sample, emb_weight):
    b, s = neighbors.shape
    idx = neighbors.astype(jnp.int32)

    info = plsc.get_sparse_core_info()
    nw = info.num_cores * info.num_subcores
    rpc = max(1, 128 // s)
    align = nw * rpc * 2
    while align % (nw * 8):
        align *= 2
    bp = ((b + align - 1) // align) * align
    if bp != b:
        idx = jnp.pad(idx, ((0, bp - b), (0, 0)))
    nb = bp // nw
    nchunks = nb // rpc
    idx3 = idx.reshape(nw, nchunks, rpc * s)

    out = _gather_mean(idx3, emb_weight.astype(jnp.float32),
                       nb, nchunks, rpc, s)
    return out[:b]
